# baseline (speedup 1.0000x reference)
import functools
import math

import jax
import jax.numpy as jnp
import numpy as np
from jax import lax
from jax.experimental import pallas as pl
from jax.experimental.pallas import tpu as pltpu
from jax.experimental.pallas import tpu_sc as plsc

_DIM = 128
_BITS = 4
_N_LEVELS = 2 ** _BITS


def _lm_levels(bits: int, n_iter: int = 200) -> np.ndarray:
    n = 2 ** bits
    xs = np.linspace(-10.0, 10.0, 400001)
    pdf = np.exp(-np.abs(xs) * math.sqrt(2.0))
    levels = np.linspace(-3.0, 3.0, n)
    for _ in range(n_iter):
        b = (levels[:-1] + levels[1:]) / 2.0
        idx = np.searchsorted(b, xs)
        num = np.bincount(idx, weights=pdf * xs, minlength=n)
        den = np.bincount(idx, weights=pdf, minlength=n)
        levels = np.where(den > 1e-12, num / np.maximum(den, 1e-12), levels)
    return np.sort(levels).astype(np.float32)


_LEVELS = _lm_levels(_BITS)
_BOUNDS = ((_LEVELS[:-1] + _LEVELS[1:]) / 2.0).astype(np.float32)

_NC, _NS, _L = 2, 16, 16
_NW = _NC * _NS
_CH = 128


def _rsqrt_newton(v):
    i = lax.bitcast_convert_type(v, jnp.int32)
    i = jnp.int32(0x5F3759DF) - lax.shift_right_logical(i, 1)
    y = lax.bitcast_convert_type(i, jnp.float32)
    for _ in range(4):
        y = y * (1.5 - 0.5 * v * y * y)
    return y


def _make_sc_kernel(rows):
    rows_per_w = rows // _NW
    n_chunks = rows_per_w // _CH
    n_groups = _CH // _L
    mesh = plsc.VectorSubcoreMesh(core_axis_name="c", subcore_axis_name="s")

    @functools.partial(
        pl.kernel,
        out_type=jax.ShapeDtypeStruct((rows, _DIM), jnp.float32),
        mesh=mesh,
        scratch_types=[
            pltpu.VMEM((_CH, _DIM), jnp.float32),
            pltpu.VMEM((_CH, _DIM), jnp.float32),
        ],
        compiler_params=pltpu.CompilerParams(needs_layout_passes=False),
    )
    def sc_kernel(x_hbm, out_hbm, xv, ov):
        wid = lax.axis_index("s") * _NC + lax.axis_index("c")
        lanes = lax.broadcasted_iota(jnp.int32, (_L,), 0)

        def chunk_body(ci, _):
            base = (wid * n_chunks + ci) * _CH
            pltpu.sync_copy(x_hbm.at[pl.ds(base, _CH)], xv)

            def group_body(g, _):
                rr = g * _L + lanes

                def stats(j, carry):
                    m, sm, sq = carry
                    jv = jnp.full((_L,), j, jnp.int32)
                    t = plsc.load_gather(xv, [rr, jv])
                    return jnp.minimum(m, t), sm + t, sq + t * t

                big = jnp.full((_L,), jnp.inf, jnp.float32)
                zero = jnp.zeros((_L,), jnp.float32)
                m, sm, sq = lax.fori_loop(0, _DIM, stats, (big, zero, zero))

                ssq = sq - (2.0 * m) * sm + jnp.float32(_DIM) * (m * m)
                ssq = jnp.maximum(ssq, 0.0) + 1e-35
                y = _rsqrt_newton(ssq)
                s = ssq * y * (1.0 / math.sqrt(_DIM)) + 1e-10
                tb = [jnp.float32(_BOUNDS[b]) * s for b in range(7, 15)]

                def quant(j, carry):
                    num, den = carry
                    jv = jnp.full((_L,), j, jnp.int32)
                    t = plsc.load_gather(xv, [rr, jv])
                    xc = t - m
                    recon = jnp.full((_L,), _LEVELS[7], jnp.float32)
                    for k in range(8):
                        recon = jnp.where(xc > tb[k], jnp.float32(_LEVELS[8 + k]), recon)
                    plsc.store_scatter(ov, [rr, jv], recon)
                    return num + xc * recon, den + recon * recon

                num, den = lax.fori_loop(0, _DIM, quant, (zero, zero))
                gamma = num / den

                def finish(j, _):
                    jv = jnp.full((_L,), j, jnp.int32)
                    r = plsc.load_gather(ov, [rr, jv])
                    plsc.store_scatter(ov, [rr, jv], r * gamma + m)
                    return 0

                lax.fori_loop(0, _DIM, finish, 0)
                return 0

            lax.fori_loop(0, n_groups, group_body, 0)
            pltpu.sync_copy(ov, out_hbm.at[pl.ds(base, _CH)])
            return 0

        lax.fori_loop(0, n_chunks, chunk_body, 0)

    return sc_kernel


def kernel(x):
    shape = x.shape
    rows = math.prod(shape[:-1])
    xf = x.reshape(rows, _DIM)
    out = _make_sc_kernel(rows)(xf)
    return out.reshape(shape)


# speedup vs baseline: 1.7076x; 1.7076x over previous
import functools
import math

import jax
import jax.numpy as jnp
import numpy as np
from jax import lax
from jax.experimental import pallas as pl
from jax.experimental.pallas import tpu as pltpu
from jax.experimental.pallas import tpu_sc as plsc

_DIM = 128
_BITS = 4
_N_LEVELS = 2 ** _BITS


def _lm_levels(bits: int, n_iter: int = 200) -> np.ndarray:
    n = 2 ** bits
    xs = np.linspace(-10.0, 10.0, 400001)
    pdf = np.exp(-np.abs(xs) * math.sqrt(2.0))
    levels = np.linspace(-3.0, 3.0, n)
    for _ in range(n_iter):
        b = (levels[:-1] + levels[1:]) / 2.0
        idx = np.searchsorted(b, xs)
        num = np.bincount(idx, weights=pdf * xs, minlength=n)
        den = np.bincount(idx, weights=pdf, minlength=n)
        levels = np.where(den > 1e-12, num / np.maximum(den, 1e-12), levels)
    return np.sort(levels).astype(np.float32)


_LEVELS = _lm_levels(_BITS)
_BOUNDS = ((_LEVELS[:-1] + _LEVELS[1:]) / 2.0).astype(np.float32)

_NC, _NS, _L = 2, 16, 16
_NW = _NC * _NS
_CH = 128
_UNROLL = 8
_NACC = 4


def _rsqrt_newton(v):
    i = lax.bitcast_convert_type(v, jnp.int32)
    i = jnp.int32(0x5F3759DF) - lax.shift_right_logical(i, 1)
    y = lax.bitcast_convert_type(i, jnp.float32)
    for _ in range(4):
        y = y * (1.5 - 0.5 * v * y * y)
    return y


def _make_sc_kernel(rows):
    rows_per_w = rows // _NW
    n_chunks = rows_per_w // _CH
    n_groups = _CH // _L
    mesh = plsc.VectorSubcoreMesh(core_axis_name="c", subcore_axis_name="s")

    @functools.partial(
        pl.kernel,
        out_type=jax.ShapeDtypeStruct((rows, _DIM), jnp.float32),
        mesh=mesh,
        scratch_types=[
            pltpu.VMEM((_CH, _DIM), jnp.float32),
            pltpu.VMEM((_CH, _DIM), jnp.float32),
        ],
        compiler_params=pltpu.CompilerParams(needs_layout_passes=False),
    )
    def sc_kernel(x_hbm, out_hbm, xv, ov):
        wid = lax.axis_index("s") * _NC + lax.axis_index("c")
        lanes = lax.broadcasted_iota(jnp.int32, (_L,), 0)

        def chunk_body(ci, _):
            base = (wid * n_chunks + ci) * _CH
            pltpu.sync_copy(x_hbm.at[pl.ds(base, _CH)], xv)

            def group_body(g, _):
                rr = g * _L + lanes
                big = jnp.full((_L,), jnp.inf, jnp.float32)
                zero = jnp.zeros((_L,), jnp.float32)

                def stats(jj, carry):
                    ms, sms, sqs = map(list, carry)
                    ts = [plsc.load_gather(xv, [rr, jnp.full((_L,), jj + u, jnp.int32)])
                          for u in range(_UNROLL)]
                    for u in range(_UNROLL):
                        p = u % _NACC
                        ms[p] = jnp.minimum(ms[p], ts[u])
                        sms[p] = sms[p] + ts[u]
                        sqs[p] = sqs[p] + ts[u] * ts[u]
                    return tuple(ms), tuple(sms), tuple(sqs)

                carry0 = ((big,) * _NACC, (zero,) * _NACC, (zero,) * _NACC)
                ms, sms, sqs = plsc.parallel_loop(
                    0, _DIM, step=_UNROLL, carry=carry0)(stats)
                m = jnp.minimum(jnp.minimum(ms[0], ms[1]),
                                jnp.minimum(ms[2], ms[3]))
                sm = (sms[0] + sms[1]) + (sms[2] + sms[3])
                sq = (sqs[0] + sqs[1]) + (sqs[2] + sqs[3])

                ssq = sq - (2.0 * m) * sm + jnp.float32(_DIM) * (m * m)
                ssq = jnp.maximum(ssq, 0.0) + 1e-35
                y = _rsqrt_newton(ssq)
                s = ssq * y * (1.0 / math.sqrt(_DIM)) + 1e-10
                tb = [jnp.float32(_BOUNDS[b]) * s for b in range(7, 15)]

                def quant(jj, carry):
                    nums, dens = map(list, carry)
                    jvs = [jnp.full((_L,), jj + u, jnp.int32)
                           for u in range(_UNROLL)]
                    ts = [plsc.load_gather(xv, [rr, jvs[u]])
                          for u in range(_UNROLL)]
                    for u in range(_UNROLL):
                        xc = ts[u] - m
                        recon = jnp.full((_L,), _LEVELS[7], jnp.float32)
                        for k in range(8):
                            recon = jnp.where(xc > tb[k],
                                              jnp.float32(_LEVELS[8 + k]), recon)
                        plsc.store_scatter(ov, [rr, jvs[u]], recon)
                        p = u % _NACC
                        nums[p] = nums[p] + xc * recon
                        dens[p] = dens[p] + recon * recon
                    return tuple(nums), tuple(dens)

                nums, dens = plsc.parallel_loop(
                    0, _DIM, step=_UNROLL,
                    carry=((zero,) * _NACC, (zero,) * _NACC))(quant)
                num = (nums[0] + nums[1]) + (nums[2] + nums[3])
                den = (dens[0] + dens[1]) + (dens[2] + dens[3])
                gamma = num / den

                def finish(jj):
                    for u in range(_UNROLL):
                        jv = jnp.full((_L,), jj + u, jnp.int32)
                        r = plsc.load_gather(ov, [rr, jv])
                        plsc.store_scatter(ov, [rr, jv], r * gamma + m)

                plsc.parallel_loop(0, _DIM, step=_UNROLL)(finish)
                return 0

            lax.fori_loop(0, n_groups, group_body, 0)
            pltpu.sync_copy(ov, out_hbm.at[pl.ds(base, _CH)])
            return 0

        lax.fori_loop(0, n_chunks, chunk_body, 0)

    return sc_kernel


def kernel(x):
    shape = x.shape
    rows = math.prod(shape[:-1])
    xf = x.reshape(rows, _DIM)
    out = _make_sc_kernel(rows)(xf)
    return out.reshape(shape)


# speedup vs baseline: 6.1544x; 3.6042x over previous
import math

import jax
import jax.numpy as jnp
import numpy as np
from jax.experimental import pallas as pl
from jax.experimental.pallas import tpu as pltpu

_DIM = 128
_BITS = 4
_N_LEVELS = 2 ** _BITS


def _lm_levels(bits: int, n_iter: int = 200) -> np.ndarray:
    n = 2 ** bits
    xs = np.linspace(-10.0, 10.0, 400001)
    pdf = np.exp(-np.abs(xs) * math.sqrt(2.0))
    levels = np.linspace(-3.0, 3.0, n)
    for _ in range(n_iter):
        b = (levels[:-1] + levels[1:]) / 2.0
        idx = np.searchsorted(b, xs)
        num = np.bincount(idx, weights=pdf * xs, minlength=n)
        den = np.bincount(idx, weights=pdf, minlength=n)
        levels = np.where(den > 1e-12, num / np.maximum(den, 1e-12), levels)
    return np.sort(levels).astype(np.float32)


_LEVELS = _lm_levels(_BITS)
_BOUNDS = ((_LEVELS[:-1] + _LEVELS[1:]) / 2.0).astype(np.float32)

_BLOCK_ROWS = 8192
_TILE = 128


def _body(x_ref, o_ref):
    def tile(i, _):
        r0 = i * _TILE
        xb = x_ref[pl.ds(r0, _TILE), :]
        vmin = jnp.min(xb, axis=1, keepdims=True)
        xc = xb - vmin
        ssq = jnp.sum(xc * xc, axis=1, keepdims=True)
        rinv = jax.lax.rsqrt(ssq + 1e-35) * math.sqrt(_DIM)
        xu = xc * rinv
        recon = jnp.full_like(xb, _LEVELS[7])
        for b in range(7, _N_LEVELS - 1):
            recon = jnp.where(xu > _BOUNDS[b], _LEVELS[b + 1], recon)
        num = jnp.sum(xc * recon, axis=1, keepdims=True)
        den = jnp.sum(recon * recon, axis=1, keepdims=True)
        gamma = num / den
        o_ref[pl.ds(r0, _TILE), :] = recon * gamma + vmin
        return 0

    jax.lax.fori_loop(0, _BLOCK_ROWS // _TILE, tile, 0)


def kernel(x):
    shape = x.shape
    rows = math.prod(shape[:-1])
    x2 = x.reshape(rows, _DIM)
    grid = rows // _BLOCK_ROWS
    out = pl.pallas_call(
        _body,
        grid=(grid,),
        in_specs=[pl.BlockSpec((_BLOCK_ROWS, _DIM), lambda i: (i, 0))],
        out_specs=pl.BlockSpec((_BLOCK_ROWS, _DIM), lambda i: (i, 0)),
        out_shape=jax.ShapeDtypeStruct((rows, _DIM), jnp.float32),
    )(x2)
    return out.reshape(shape)


# speedup vs baseline: 17.4478x; 2.8350x over previous
import math

import jax
import jax.numpy as jnp
import numpy as np
from jax.experimental import pallas as pl
from jax.experimental.pallas import tpu as pltpu

_DIM = 128
_BITS = 4
_N_LEVELS = 2 ** _BITS


def _lm_levels(bits: int, n_iter: int = 200) -> np.ndarray:
    n = 2 ** bits
    xs = np.linspace(-10.0, 10.0, 400001)
    pdf = np.exp(-np.abs(xs) * math.sqrt(2.0))
    levels = np.linspace(-3.0, 3.0, n)
    for _ in range(n_iter):
        b = (levels[:-1] + levels[1:]) / 2.0
        idx = np.searchsorted(b, xs)
        num = np.bincount(idx, weights=pdf * xs, minlength=n)
        den = np.bincount(idx, weights=pdf, minlength=n)
        levels = np.where(den > 1e-12, num / np.maximum(den, 1e-12), levels)
    return np.sort(levels).astype(np.float32)


_LEVELS = _lm_levels(_BITS)
_BOUNDS = ((_LEVELS[:-1] + _LEVELS[1:]) / 2.0).astype(np.float32)

_BLOCK_ROWS = 16384


def _body(x_ref, o_ref):
    xb = x_ref[...]
    vmin = jnp.min(xb, axis=1, keepdims=True)
    xc = xb - vmin
    ssq = jnp.sum(xc * xc, axis=1, keepdims=True)
    rinv = jax.lax.rsqrt(ssq + 1e-35) * math.sqrt(_DIM)
    xu = xc * rinv
    recon = jnp.full_like(xb, _LEVELS[7])
    for b in range(7, _N_LEVELS - 1):
        recon = jnp.where(xu > _BOUNDS[b], _LEVELS[b + 1], recon)
    num = jnp.sum(xc * recon, axis=1, keepdims=True)
    den = jnp.sum(recon * recon, axis=1, keepdims=True)
    gamma = num / den
    o_ref[...] = recon * gamma + vmin


def kernel(x):
    shape = x.shape
    rows = math.prod(shape[:-1])
    x2 = x.reshape(rows, _DIM)
    grid = rows // _BLOCK_ROWS
    out = pl.pallas_call(
        _body,
        grid=(grid,),
        in_specs=[pl.BlockSpec((_BLOCK_ROWS, _DIM), lambda i: (i, 0))],
        out_specs=pl.BlockSpec((_BLOCK_ROWS, _DIM), lambda i: (i, 0)),
        out_shape=jax.ShapeDtypeStruct((rows, _DIM), jnp.float32),
    )(x2)
    return out.reshape(shape)


# speedup vs baseline: 17.8243x; 1.0216x over previous
import math

import jax
import jax.numpy as jnp
import numpy as np
from jax.experimental import pallas as pl

_DIM = 128
_BITS = 4
_N_LEVELS = 2 ** _BITS


def _lm_levels(bits: int, n_iter: int = 200) -> np.ndarray:
    n = 2 ** bits
    xs = np.linspace(-10.0, 10.0, 400001)
    pdf = np.exp(-np.abs(xs) * math.sqrt(2.0))
    levels = np.linspace(-3.0, 3.0, n)
    for _ in range(n_iter):
        b = (levels[:-1] + levels[1:]) / 2.0
        idx = np.searchsorted(b, xs)
        num = np.bincount(idx, weights=pdf * xs, minlength=n)
        den = np.bincount(idx, weights=pdf, minlength=n)
        levels = np.where(den > 1e-12, num / np.maximum(den, 1e-12), levels)
    return np.sort(levels).astype(np.float32)


_LEVELS = _lm_levels(_BITS)
_BOUNDS = ((_LEVELS[:-1] + _LEVELS[1:]) / 2.0).astype(np.float32)

_BLOCK_ROWS = 8192


def _body(x_ref, o_ref):
    xb = x_ref[...]
    vmin = jnp.min(xb, axis=1, keepdims=True)
    xc = xb - vmin
    ssq = jnp.sum(xc * xc, axis=1, keepdims=True)
    rinv = jax.lax.rsqrt(ssq + 1e-35) * math.sqrt(_DIM)
    xu = xc * rinv
    recon = jnp.full_like(xb, _LEVELS[7])
    for b in range(7, _N_LEVELS - 1):
        recon = jnp.where(xu > _BOUNDS[b], _LEVELS[b + 1], recon)
    num = jnp.sum(xc * recon, axis=1, keepdims=True)
    den = jnp.sum(recon * recon, axis=1, keepdims=True)
    gamma = num / den
    o_ref[...] = recon * gamma + vmin


def kernel(x):
    shape = x.shape
    rows = math.prod(shape[:-1])
    x2 = x.reshape(rows, _DIM)
    grid = rows // _BLOCK_ROWS
    out = pl.pallas_call(
        _body,
        grid=(grid,),
        in_specs=[pl.BlockSpec((_BLOCK_ROWS, _DIM), lambda i: (i, 0))],
        out_specs=pl.BlockSpec((_BLOCK_ROWS, _DIM), lambda i: (i, 0)),
        out_shape=jax.ShapeDtypeStruct((rows, _DIM), jnp.float32),
    )(x2)
    return out.reshape(shape)
